# Initial kernel scaffold; baseline (speedup 1.0000x reference)
import jax
import jax.numpy as jnp
from jax.experimental import pallas as pl


def kernel(x, W_proj, b_proj, sim_matrix, temperature):
    raise NotImplementedError("write your pallas kernel here")


# speedup vs baseline: 4.8370x; 4.8370x over previous
import functools

import jax
import jax.numpy as jnp
import numpy as np
from jax.experimental import pallas as pl
from jax.experimental.pallas import tpu as pltpu

_NUM_EXPERTS = 64
_CLAMP_MAX = float(np.log(100.0))


def _router_body(x_ref, w_ref, b_ref, sim_ref, temp_ref,
                 mask_ref, prob_ref, imp_ref, load_ref):
    x_t = x_ref[...]
    w = w_ref[...]
    proj = jax.lax.dot_general(
        x_t, w, (((1,), (1,)), ((), ())),
        preferred_element_type=jnp.float32) + b_ref[...]

    norm = jnp.sqrt(jnp.sum(proj * proj, axis=1, keepdims=True))
    projn = proj / jnp.maximum(norm, 1e-12)

    s = sim_ref[...]
    s_norm = jnp.sqrt(jnp.sum(s * s, axis=0, keepdims=True))
    sn = s / jnp.maximum(s_norm, 1e-12)

    scale = jnp.exp(jnp.minimum(temp_ref[0, 0], _CLAMP_MAX))
    logits = jnp.dot(projn, sn, preferred_element_type=jnp.float32) * scale

    iota = jax.lax.broadcasted_iota(jnp.int32, logits.shape, 1)
    m1 = jnp.max(logits, axis=1, keepdims=True)
    i1 = jnp.min(jnp.where(logits == m1, iota, _NUM_EXPERTS),
                 axis=1, keepdims=True)
    is1 = iota == i1
    l2 = jnp.where(is1, -jnp.inf, logits)
    m2 = jnp.max(l2, axis=1, keepdims=True)
    i2 = jnp.min(jnp.where(l2 == m2, iota, _NUM_EXPERTS),
                 axis=1, keepdims=True)
    mask = (is1 | (iota == i2)).astype(jnp.float32)

    p = jnp.exp(logits - m1)
    p = p / jnp.sum(p, axis=1, keepdims=True)

    mask_ref[...] = mask
    prob_ref[...] = p

    imp_part = jnp.broadcast_to(jnp.sum(p, axis=0, keepdims=True), (8, _NUM_EXPERTS))
    load_part = jnp.broadcast_to(jnp.sum(mask, axis=0, keepdims=True), (8, _NUM_EXPERTS))

    @pl.when(pl.program_id(0) == 0)
    def _init():
        imp_ref[...] = imp_part
        load_ref[...] = load_part

    @pl.when(pl.program_id(0) != 0)
    def _accum():
        imp_ref[...] += imp_part
        load_ref[...] += load_part


@functools.partial(jax.jit, static_argnames=())
def kernel(x, W_proj, b_proj, sim_matrix, temperature):
    n, d = x.shape
    e = sim_matrix.shape[1]
    tile = 1024
    while n % tile:
        tile //= 2
    grid = (n // tile,)

    b2 = b_proj.reshape(1, e)
    t2 = temperature.reshape(1, 1)

    mask, prob, imp, load = pl.pallas_call(
        _router_body,
        grid=grid,
        in_specs=[
            pl.BlockSpec((tile, d), lambda i: (i, 0)),
            pl.BlockSpec((e, d), lambda i: (0, 0)),
            pl.BlockSpec((1, e), lambda i: (0, 0)),
            pl.BlockSpec((sim_matrix.shape[0], e), lambda i: (0, 0)),
            pl.BlockSpec((1, 1), lambda i: (0, 0)),
        ],
        out_specs=[
            pl.BlockSpec((tile, e), lambda i: (i, 0)),
            pl.BlockSpec((tile, e), lambda i: (i, 0)),
            pl.BlockSpec((8, e), lambda i: (0, 0)),
            pl.BlockSpec((8, e), lambda i: (0, 0)),
        ],
        out_shape=[
            jax.ShapeDtypeStruct((n, e), jnp.float32),
            jax.ShapeDtypeStruct((n, e), jnp.float32),
            jax.ShapeDtypeStruct((8, e), jnp.float32),
            jax.ShapeDtypeStruct((8, e), jnp.float32),
        ],
        compiler_params=pltpu.CompilerParams(
            dimension_semantics=("arbitrary",)),
    )(x, W_proj, b2, sim_matrix, t2)

    return (mask, prob, imp[0], load[0])


# speedup vs baseline: 5.3143x; 1.0987x over previous
import functools

import jax
import jax.numpy as jnp
import numpy as np
from jax.experimental import pallas as pl
from jax.experimental.pallas import tpu as pltpu

_NUM_EXPERTS = 64
_CLAMP_MAX = float(np.log(100.0))


def _router_body(x_ref, w_ref, b_ref, sim_ref, temp_ref,
                 mask_ref, prob_ref, imp_ref, load_ref):
    x_t = x_ref[...]
    w = w_ref[...]
    proj = jax.lax.dot_general(
        x_t, w, (((1,), (1,)), ((), ())),
        preferred_element_type=jnp.float32) + b_ref[...]

    norm = jnp.sqrt(jnp.sum(proj * proj, axis=1, keepdims=True))
    projn = proj / jnp.maximum(norm, 1e-12)

    s = sim_ref[...]
    s_norm = jnp.sqrt(jnp.sum(s * s, axis=0, keepdims=True))
    sn = s / jnp.maximum(s_norm, 1e-12)

    scale = jnp.exp(jnp.minimum(temp_ref[0, 0], _CLAMP_MAX))
    logits = jnp.dot(projn, sn, preferred_element_type=jnp.float32) * scale

    m1 = jnp.max(logits, axis=1, keepdims=True)
    l2 = jnp.where(logits == m1, -jnp.inf, logits)
    m2 = jnp.max(l2, axis=1, keepdims=True)
    mask = (logits >= m2).astype(jnp.float32)

    p = jnp.exp(logits - m1)
    p = p / jnp.sum(p, axis=1, keepdims=True)

    mask_ref[...] = mask
    prob_ref[...] = p

    imp_part = jnp.broadcast_to(jnp.sum(p, axis=0, keepdims=True), (8, _NUM_EXPERTS))
    load_part = jnp.broadcast_to(jnp.sum(mask, axis=0, keepdims=True), (8, _NUM_EXPERTS))

    @pl.when(pl.program_id(0) == 0)
    def _init():
        imp_ref[...] = imp_part
        load_ref[...] = load_part

    @pl.when(pl.program_id(0) != 0)
    def _accum():
        imp_ref[...] += imp_part
        load_ref[...] += load_part


@functools.partial(jax.jit, static_argnames=())
def kernel(x, W_proj, b_proj, sim_matrix, temperature):
    n, d = x.shape
    e = sim_matrix.shape[1]
    tile = 1024
    while n % tile:
        tile //= 2
    grid = (n // tile,)

    b2 = b_proj.reshape(1, e)
    t2 = temperature.reshape(1, 1)

    mask, prob, imp, load = pl.pallas_call(
        _router_body,
        grid=grid,
        in_specs=[
            pl.BlockSpec((tile, d), lambda i: (i, 0)),
            pl.BlockSpec((e, d), lambda i: (0, 0)),
            pl.BlockSpec((1, e), lambda i: (0, 0)),
            pl.BlockSpec((sim_matrix.shape[0], e), lambda i: (0, 0)),
            pl.BlockSpec((1, 1), lambda i: (0, 0)),
        ],
        out_specs=[
            pl.BlockSpec((tile, e), lambda i: (i, 0)),
            pl.BlockSpec((tile, e), lambda i: (i, 0)),
            pl.BlockSpec((8, e), lambda i: (0, 0)),
            pl.BlockSpec((8, e), lambda i: (0, 0)),
        ],
        out_shape=[
            jax.ShapeDtypeStruct((n, e), jnp.float32),
            jax.ShapeDtypeStruct((n, e), jnp.float32),
            jax.ShapeDtypeStruct((8, e), jnp.float32),
            jax.ShapeDtypeStruct((8, e), jnp.float32),
        ],
        compiler_params=pltpu.CompilerParams(
            dimension_semantics=("arbitrary",)),
    )(x, W_proj, b2, sim_matrix, t2)

    return (mask, prob, imp[0], load[0])


# speedup vs baseline: 6.0506x; 1.1385x over previous
import functools

import jax
import jax.numpy as jnp
import numpy as np
from jax.experimental import pallas as pl
from jax.experimental.pallas import tpu as pltpu

_NUM_EXPERTS = 64
_CLAMP_MAX = float(np.log(100.0))


def _router_body(x_ref, w_ref, b_ref, sim_ref, temp_ref,
                 mask_ref, prob_ref, imp_ref, load_ref):
    x_t = x_ref[...]
    w = w_ref[...]
    proj = jax.lax.dot_general(
        x_t, w, (((1,), (1,)), ((), ())),
        preferred_element_type=jnp.float32) + b_ref[...]

    norm = jnp.sqrt(jnp.sum(proj * proj, axis=1, keepdims=True))
    projn = proj / jnp.maximum(norm, 1e-12)

    s = sim_ref[...]
    s_norm = jnp.sqrt(jnp.sum(s * s, axis=0, keepdims=True))
    sn = s / jnp.maximum(s_norm, 1e-12)

    scale = jnp.exp(jnp.minimum(temp_ref[0, 0], _CLAMP_MAX))
    logits = jnp.dot(projn, sn, preferred_element_type=jnp.float32) * scale

    m1 = jnp.max(logits, axis=1, keepdims=True)
    l2 = jnp.where(logits == m1, -jnp.inf, logits)
    m2 = jnp.max(l2, axis=1, keepdims=True)
    mask = (logits >= m2).astype(jnp.float32)

    p = jnp.exp(logits - m1)
    p = p / jnp.sum(p, axis=1, keepdims=True)

    mask_ref[...] = mask
    prob_ref[...] = p

    imp_part = jnp.broadcast_to(jnp.sum(p, axis=0, keepdims=True), (8, _NUM_EXPERTS))
    load_part = jnp.broadcast_to(jnp.sum(mask, axis=0, keepdims=True), (8, _NUM_EXPERTS))

    @pl.when(pl.program_id(0) == 0)
    def _init():
        imp_ref[...] = imp_part
        load_ref[...] = load_part

    @pl.when(pl.program_id(0) != 0)
    def _accum():
        imp_ref[...] += imp_part
        load_ref[...] += load_part


@functools.partial(jax.jit, static_argnames=())
def kernel(x, W_proj, b_proj, sim_matrix, temperature):
    n, d = x.shape
    e = sim_matrix.shape[1]
    tile = 2048
    while n % tile:
        tile //= 2
    grid = (n // tile,)

    b2 = b_proj.reshape(1, e)
    t2 = temperature.reshape(1, 1)

    mask, prob, imp, load = pl.pallas_call(
        _router_body,
        grid=grid,
        in_specs=[
            pl.BlockSpec((tile, d), lambda i: (i, 0)),
            pl.BlockSpec((e, d), lambda i: (0, 0)),
            pl.BlockSpec((1, e), lambda i: (0, 0)),
            pl.BlockSpec((sim_matrix.shape[0], e), lambda i: (0, 0)),
            pl.BlockSpec((1, 1), lambda i: (0, 0)),
        ],
        out_specs=[
            pl.BlockSpec((tile, e), lambda i: (i, 0)),
            pl.BlockSpec((tile, e), lambda i: (i, 0)),
            pl.BlockSpec((8, e), lambda i: (0, 0)),
            pl.BlockSpec((8, e), lambda i: (0, 0)),
        ],
        out_shape=[
            jax.ShapeDtypeStruct((n, e), jnp.float32),
            jax.ShapeDtypeStruct((n, e), jnp.float32),
            jax.ShapeDtypeStruct((8, e), jnp.float32),
            jax.ShapeDtypeStruct((8, e), jnp.float32),
        ],
        compiler_params=pltpu.CompilerParams(
            dimension_semantics=("arbitrary",)),
    )(x, W_proj, b2, sim_matrix, t2)

    return (mask, prob, imp[0], load[0])


# speedup vs baseline: 6.2354x; 1.0305x over previous
import functools

import jax
import jax.numpy as jnp
import numpy as np
from jax.experimental import pallas as pl
from jax.experimental.pallas import tpu as pltpu

_NUM_EXPERTS = 64
_CLAMP_MAX = float(np.log(100.0))


def _router_body(x_ref, w_ref, b_ref, sim_ref, temp_ref,
                 mask_ref, prob_ref, imp_ref, load_ref):
    x_t = x_ref[...]
    w = w_ref[...]
    proj = jax.lax.dot_general(
        x_t, w, (((1,), (1,)), ((), ())),
        preferred_element_type=jnp.float32) + b_ref[...]

    norm = jnp.sqrt(jnp.sum(proj * proj, axis=1, keepdims=True))
    projn = proj / jnp.maximum(norm, 1e-12)

    s = sim_ref[...]
    s_norm = jnp.sqrt(jnp.sum(s * s, axis=0, keepdims=True))
    sn = s / jnp.maximum(s_norm, 1e-12)

    scale = jnp.exp(jnp.minimum(temp_ref[0, 0], _CLAMP_MAX))
    logits = jnp.dot(projn, sn, preferred_element_type=jnp.float32) * scale

    m1 = jnp.max(logits, axis=1, keepdims=True)
    l2 = jnp.where(logits == m1, -jnp.inf, logits)
    m2 = jnp.max(l2, axis=1, keepdims=True)
    mask = (logits >= m2).astype(jnp.float32)

    p = jnp.exp(logits - m1)
    p = p / jnp.sum(p, axis=1, keepdims=True)

    mask_ref[...] = mask
    prob_ref[...] = p

    imp_part = jnp.broadcast_to(jnp.sum(p, axis=0, keepdims=True), (8, _NUM_EXPERTS))
    load_part = jnp.broadcast_to(jnp.sum(mask, axis=0, keepdims=True), (8, _NUM_EXPERTS))

    @pl.when(pl.program_id(0) == 0)
    def _init():
        imp_ref[...] = imp_part
        load_ref[...] = load_part

    @pl.when(pl.program_id(0) != 0)
    def _accum():
        imp_ref[...] += imp_part
        load_ref[...] += load_part


@functools.partial(jax.jit, static_argnames=())
def kernel(x, W_proj, b_proj, sim_matrix, temperature):
    n, d = x.shape
    e = sim_matrix.shape[1]
    tile = 4096
    while n % tile:
        tile //= 2
    grid = (n // tile,)

    b2 = b_proj.reshape(1, e)
    t2 = temperature.reshape(1, 1)

    mask, prob, imp, load = pl.pallas_call(
        _router_body,
        grid=grid,
        in_specs=[
            pl.BlockSpec((tile, d), lambda i: (i, 0)),
            pl.BlockSpec((e, d), lambda i: (0, 0)),
            pl.BlockSpec((1, e), lambda i: (0, 0)),
            pl.BlockSpec((sim_matrix.shape[0], e), lambda i: (0, 0)),
            pl.BlockSpec((1, 1), lambda i: (0, 0)),
        ],
        out_specs=[
            pl.BlockSpec((tile, e), lambda i: (i, 0)),
            pl.BlockSpec((tile, e), lambda i: (i, 0)),
            pl.BlockSpec((8, e), lambda i: (0, 0)),
            pl.BlockSpec((8, e), lambda i: (0, 0)),
        ],
        out_shape=[
            jax.ShapeDtypeStruct((n, e), jnp.float32),
            jax.ShapeDtypeStruct((n, e), jnp.float32),
            jax.ShapeDtypeStruct((8, e), jnp.float32),
            jax.ShapeDtypeStruct((8, e), jnp.float32),
        ],
        compiler_params=pltpu.CompilerParams(
            dimension_semantics=("arbitrary",)),
    )(x, W_proj, b2, sim_matrix, t2)

    return (mask, prob, imp[0], load[0])


# speedup vs baseline: 10.7617x; 1.7259x over previous
import functools

import jax
import jax.numpy as jnp
import numpy as np
from jax.experimental import pallas as pl
from jax.experimental.pallas import tpu as pltpu

_NUM_EXPERTS = 64
_CLAMP_MAX = float(np.log(100.0))


def _router_body(x_ref, w_ref, b_ref, sim_ref, temp_ref,
                 mask_ref, prob_ref, imp_ref, load_ref):
    x_t = x_ref[...]
    w = w_ref[...]
    projT = jax.lax.dot_general(
        w, x_t, (((1,), (1,)), ((), ())),
        preferred_element_type=jnp.float32) + b_ref[...]

    norm = jnp.sqrt(jnp.sum(projT * projT, axis=0, keepdims=True))
    projn = projT / jnp.maximum(norm, 1e-12)

    s = sim_ref[...]
    s_norm = jnp.sqrt(jnp.sum(s * s, axis=0, keepdims=True))
    sn = s / jnp.maximum(s_norm, 1e-12)

    scale = jnp.exp(jnp.minimum(temp_ref[0, 0], _CLAMP_MAX))
    logits = jax.lax.dot_general(
        sn, projn, (((0,), (0,)), ((), ())),
        preferred_element_type=jnp.float32) * scale

    m1 = jnp.max(logits, axis=0, keepdims=True)
    l2 = jnp.where(logits == m1, -jnp.inf, logits)
    m2 = jnp.max(l2, axis=0, keepdims=True)
    mask = (logits >= m2).astype(jnp.float32)

    p = jnp.exp(logits - m1)
    p = p / jnp.sum(p, axis=0, keepdims=True)

    mask_ref[...] = mask
    prob_ref[...] = p

    imp_part = jnp.broadcast_to(
        jnp.sum(p, axis=1, keepdims=True), (_NUM_EXPERTS, 128))
    load_part = jnp.broadcast_to(
        jnp.sum(mask, axis=1, keepdims=True), (_NUM_EXPERTS, 128))

    @pl.when(pl.program_id(0) == 0)
    def _init():
        imp_ref[...] = imp_part
        load_ref[...] = load_part

    @pl.when(pl.program_id(0) != 0)
    def _accum():
        imp_ref[...] += imp_part
        load_ref[...] += load_part


@functools.partial(jax.jit, static_argnames=())
def kernel(x, W_proj, b_proj, sim_matrix, temperature):
    n, d = x.shape
    e = sim_matrix.shape[1]
    h = sim_matrix.shape[0]
    tile = 4096
    while n % tile:
        tile //= 2
    grid = (n // tile,)

    b2 = b_proj.reshape(e, 1)
    t2 = temperature.reshape(1, 1)

    maskT, probT, imp, load = pl.pallas_call(
        _router_body,
        grid=grid,
        in_specs=[
            pl.BlockSpec((tile, d), lambda i: (i, 0)),
            pl.BlockSpec((e, d), lambda i: (0, 0)),
            pl.BlockSpec((e, 1), lambda i: (0, 0)),
            pl.BlockSpec((h, e), lambda i: (0, 0)),
            pl.BlockSpec((1, 1), lambda i: (0, 0)),
        ],
        out_specs=[
            pl.BlockSpec((e, tile), lambda i: (0, i)),
            pl.BlockSpec((e, tile), lambda i: (0, i)),
            pl.BlockSpec((e, 128), lambda i: (0, 0)),
            pl.BlockSpec((e, 128), lambda i: (0, 0)),
        ],
        out_shape=[
            jax.ShapeDtypeStruct((e, n), jnp.float32),
            jax.ShapeDtypeStruct((e, n), jnp.float32),
            jax.ShapeDtypeStruct((e, 128), jnp.float32),
            jax.ShapeDtypeStruct((e, 128), jnp.float32),
        ],
        compiler_params=pltpu.CompilerParams(
            dimension_semantics=("arbitrary",)),
    )(x, W_proj, b2, sim_matrix, t2)

    return (maskT.T, probT.T, imp[:, 0], load[:, 0])
